# baseline (speedup 1.0000x reference)
import functools

import jax
import jax.numpy as jnp
from jax import lax
from jax.experimental import pallas as pl
from jax.experimental.pallas import tpu as pltpu
from jax.experimental.pallas import tpu_sc as plsc

_BATCH = 4096
_HIST = 200
_D = 64
_B = _BATCH * _HIST
_NC = 2
_NS = 16
_NW = _NC * _NS
_BPW = _B // _NW
_CH = 800
_NCHUNK = _BPW // _CH


def _make_kernel():
    mesh = plsc.VectorSubcoreMesh(core_axis_name="c", subcore_axis_name="s")

    @functools.partial(
        pl.kernel,
        out_type=jax.ShapeDtypeStruct((_B, _D), jnp.float32),
        mesh=mesh,
        scratch_types=[
            pltpu.VMEM((_CH,), jnp.int32),
            pltpu.VMEM((_CH, _D), jnp.float32),
            pltpu.SemaphoreType.DMA,
        ],
        compiler_params=pltpu.CompilerParams(use_tc_tiling_on_sc=False),
    )
    def emb(idx_hbm, table_hbm, out_hbm, idx_v, rows_v, sem):
        wid = lax.axis_index("s") * _NC + lax.axis_index("c")
        base = wid * _BPW

        @pl.loop(0, _NCHUNK)
        def _chunk(i):
            off = base + i * _CH
            pltpu.sync_copy(idx_hbm.at[pl.ds(off, _CH)], idx_v)
            pltpu.async_copy(table_hbm.at[idx_v], rows_v, sem).wait()
            pltpu.sync_copy(rows_v, out_hbm.at[pl.ds(off, _CH)])

    return emb


_emb = _make_kernel()


@jax.jit
def kernel(x, table):
    idx = x.reshape(_B).astype(jnp.int32)
    out = _emb(idx, table)
    return out.reshape(_BATCH, _HIST, _D)


# speedup vs baseline: 1.0220x; 1.0220x over previous
import functools

import jax
import jax.numpy as jnp
from jax import lax
from jax.experimental import pallas as pl
from jax.experimental.pallas import tpu as pltpu
from jax.experimental.pallas import tpu_sc as plsc

_BATCH = 4096
_HIST = 200
_D = 64
_B = _BATCH * _HIST
_NC = 2
_NS = 16
_NW = _NC * _NS
_BPW = _B // _NW
_CH = 400
_NBUF = 4
_NCHUNK = _BPW // _CH
_NGRP = _NCHUNK // _NBUF - 1


def _make_kernel():
    mesh = plsc.VectorSubcoreMesh(core_axis_name="c", subcore_axis_name="s")

    @functools.partial(
        pl.kernel,
        out_type=jax.ShapeDtypeStruct((_B, _D), jnp.float32),
        mesh=mesh,
        scratch_types=(
            [pltpu.VMEM((_BPW,), jnp.int32),
             pltpu.VMEM((_NBUF, _CH, _D), jnp.float32)]
            + [pltpu.SemaphoreType.DMA] * (2 * _NBUF)
        ),
        compiler_params=pltpu.CompilerParams(use_tc_tiling_on_sc=False),
    )
    def emb(idx_hbm, table_hbm, out_hbm, idx_v, rows_v, *sems):
        gsems = sems[:_NBUF]
        osems = sems[_NBUF:]
        wid = lax.axis_index("s") * _NC + lax.axis_index("c")
        base = wid * _BPW

        pltpu.sync_copy(idx_hbm.at[pl.ds(base, _BPW)], idx_v)

        def start_gather(i, b):
            pltpu.async_copy(
                table_hbm.at[idx_v.at[pl.ds(i * _CH, _CH)]],
                rows_v.at[b], gsems[b])

        def wait_gather(b):
            pltpu.make_async_copy(
                table_hbm.at[idx_v.at[pl.ds(0, _CH)]],
                rows_v.at[b], gsems[b]).wait()

        def start_store(i, b):
            pltpu.async_copy(
                rows_v.at[b], out_hbm.at[pl.ds(base + i * _CH, _CH)],
                osems[b])

        def wait_store(b):
            pltpu.make_async_copy(
                rows_v.at[b], out_hbm.at[pl.ds(base, _CH)], osems[b]).wait()

        for b in range(_NBUF):
            start_gather(b, b)

        @pl.loop(0, _NGRP)
        def _grp(g):
            i0 = g * _NBUF
            for b in range(_NBUF):
                wait_gather(b)
                start_store(i0 + b, b)
            for b in range(_NBUF):
                wait_store(b)
                start_gather(i0 + b + _NBUF, b)

        i0 = _NGRP * _NBUF
        for b in range(_NBUF):
            wait_gather(b)
            start_store(i0 + b, b)
        for b in range(_NBUF):
            wait_store(b)

    return emb


_emb = _make_kernel()


@jax.jit
def kernel(x, table):
    idx = x.reshape(_B).astype(jnp.int32)
    out = _emb(idx, table)
    return out.reshape(_BATCH, _HIST, _D)


# speedup vs baseline: 1.2503x; 1.2234x over previous
import functools

import jax
import jax.numpy as jnp
from jax import lax
from jax.experimental import pallas as pl
from jax.experimental.pallas import tpu as pltpu
from jax.experimental.pallas import tpu_sc as plsc

_BATCH = 4096
_HIST = 200
_D = 64
_DP = 128
_V = 1000000
_B = _BATCH * _HIST
_NC = 2
_NS = 16
_NW = _NC * _NS
_BPW = _B // _NW
_CH = _HIST
_NBUF = 4
_NCHUNK = _BPW // _CH
_NGRP = _NCHUNK // _NBUF - 1


def _make_kernel():
    mesh = plsc.VectorSubcoreMesh(core_axis_name="c", subcore_axis_name="s")

    @functools.partial(
        pl.kernel,
        out_type=jax.ShapeDtypeStruct((_BATCH, _HIST, _DP), jnp.float32),
        mesh=mesh,
        scratch_types=(
            [pltpu.VMEM((_BPW,), jnp.int32),
             pltpu.VMEM((_NBUF, _CH, _DP), jnp.float32)]
            + [pltpu.SemaphoreType.DMA] * (2 * _NBUF)
        ),
        compiler_params=pltpu.CompilerParams(use_tc_tiling_on_sc=True),
    )
    def emb(idx_hbm, table_hbm, out_hbm, idx_v, rows_v, *sems):
        gsems = sems[:_NBUF]
        osems = sems[_NBUF:]
        wid = lax.axis_index("s") * _NC + lax.axis_index("c")
        base = wid * _BPW
        b0 = wid * (_BATCH // _NW)

        pltpu.sync_copy(idx_hbm.at[pl.ds(base, _BPW)], idx_v)

        def start_gather(i, b):
            pltpu.async_copy(
                table_hbm.at[idx_v.at[pl.ds(i * _CH, _CH)]],
                rows_v.at[b], gsems[b])

        def wait_gather(b):
            pltpu.make_async_copy(
                table_hbm.at[idx_v.at[pl.ds(0, _CH)]],
                rows_v.at[b], gsems[b]).wait()

        def start_store(i, b):
            pltpu.async_copy(rows_v.at[b], out_hbm.at[b0 + i], osems[b])

        def wait_store(b):
            pltpu.make_async_copy(
                rows_v.at[b], out_hbm.at[b0], osems[b]).wait()

        for b in range(_NBUF):
            start_gather(b, b)

        @pl.loop(0, _NGRP)
        def _grp(g):
            i0 = g * _NBUF
            for b in range(_NBUF):
                wait_gather(b)
                start_store(i0 + b, b)
            for b in range(_NBUF):
                wait_store(b)
                start_gather(i0 + b + _NBUF, b)

        i0 = _NGRP * _NBUF
        for b in range(_NBUF):
            wait_gather(b)
            start_store(i0 + b, b)
        for b in range(_NBUF):
            wait_store(b)

    return emb


_emb = _make_kernel()


@jax.jit
def kernel(x, table):
    idx = x.reshape(_B).astype(jnp.int32)
    tab128 = jnp.pad(table, ((0, 0), (0, _DP - _D)))
    out128 = _emb(idx, tab128)
    return out128[:, :, :_D]


# speedup vs baseline: 1.4498x; 1.1595x over previous
import functools

import jax
import jax.numpy as jnp
from jax import lax
from jax.experimental import pallas as pl
from jax.experimental.pallas import tpu as pltpu
from jax.experimental.pallas import tpu_sc as plsc

_BATCH = 4096
_HIST = 200
_D = 64
_DP = 128
_V = 1000000
_B = _BATCH * _HIST
_NC = 2
_NS = 16
_NW = _NC * _NS
_BPW = _B // _NW
_CH = _HIST
_NBUF = 4
_NCHUNK = _BPW // _CH
_NGRP = _NCHUNK // _NBUF - 1


def _make_kernel():
    mesh = plsc.VectorSubcoreMesh(core_axis_name="c", subcore_axis_name="s")

    @functools.partial(
        pl.kernel,
        out_type=jax.ShapeDtypeStruct((_BATCH, _HIST, _DP), jnp.float32),
        mesh=mesh,
        scratch_types=(
            [pltpu.VMEM((_BPW,), jnp.int32),
             pltpu.VMEM((_NBUF, _CH, _D), jnp.float32)]
            + [pltpu.SemaphoreType.DMA] * (2 * _NBUF)
        ),
        compiler_params=pltpu.CompilerParams(use_tc_tiling_on_sc=False),
    )
    def emb(idx_hbm, table_hbm, out_hbm, idx_v, rows_v, *sems):
        gsems = sems[:_NBUF]
        osems = sems[_NBUF:]
        wid = lax.axis_index("s") * _NC + lax.axis_index("c")
        base = wid * _BPW
        b0 = wid * (_BATCH // _NW)

        pltpu.sync_copy(idx_hbm.at[pl.ds(base, _BPW)], idx_v)

        @pl.loop(0, _BPW // 16)
        def _dbl(j):
            idx_v[pl.ds(j * 16, 16)] = idx_v[pl.ds(j * 16, 16)] * 2

        def start_gather(i, b):
            pltpu.async_copy(
                table_hbm.at[idx_v.at[pl.ds(i * _CH, _CH)]],
                rows_v.at[b], gsems[b])

        def wait_gather(b):
            pltpu.make_async_copy(
                table_hbm.at[idx_v.at[pl.ds(0, _CH)]],
                rows_v.at[b], gsems[b]).wait()

        def start_store(i, b):
            pltpu.async_copy(
                rows_v.at[b], out_hbm.at[b0 + i, :, pl.ds(0, _D)], osems[b])

        def wait_store(b):
            pltpu.make_async_copy(
                rows_v.at[b], out_hbm.at[b0, :, pl.ds(0, _D)], osems[b]).wait()

        for b in range(_NBUF):
            start_gather(b, b)

        @pl.loop(0, _NGRP)
        def _grp(g):
            i0 = g * _NBUF
            for b in range(_NBUF):
                wait_gather(b)
                start_store(i0 + b, b)
            for b in range(_NBUF):
                wait_store(b)
                start_gather(i0 + b + _NBUF, b)

        i0 = _NGRP * _NBUF
        for b in range(_NBUF):
            wait_gather(b)
            start_store(i0 + b, b)
        for b in range(_NBUF):
            wait_store(b)

    return emb


_emb = _make_kernel()


@jax.jit
def kernel(x, table):
    idx = x.reshape(_B).astype(jnp.int32)
    tab2m = jnp.pad(table, ((0, 0), (0, _DP - _D))).reshape(2 * _V, _D)
    out128 = _emb(idx, tab2m)
    return out128[:, :, :_D]


# speedup vs baseline: 1.4609x; 1.0077x over previous
import functools

import jax
import jax.numpy as jnp
from jax import lax
from jax.experimental import pallas as pl
from jax.experimental.pallas import tpu as pltpu
from jax.experimental.pallas import tpu_sc as plsc

_BATCH = 4096
_HIST = 200
_D = 64
_DP = 128
_V = 1000000
_B = _BATCH * _HIST
_NC = 2
_NS = 16
_NW = _NC * _NS
_BPW = _B // _NW
_CH = _HIST
_NBUF = 8
_NCHUNK = _BPW // _CH
_NGRP = _NCHUNK // _NBUF - 1


def _make_kernel():
    mesh = plsc.VectorSubcoreMesh(core_axis_name="c", subcore_axis_name="s")

    @functools.partial(
        pl.kernel,
        out_type=jax.ShapeDtypeStruct((_BATCH, _HIST, _DP), jnp.float32),
        mesh=mesh,
        scratch_types=(
            [pltpu.VMEM((_BPW,), jnp.int32),
             pltpu.VMEM((_NBUF, _CH, _D), jnp.float32)]
            + [pltpu.SemaphoreType.DMA] * (2 * _NBUF)
        ),
        compiler_params=pltpu.CompilerParams(use_tc_tiling_on_sc=False),
    )
    def emb(idx_hbm, table_hbm, out_hbm, idx_v, rows_v, *sems):
        gsems = sems[:_NBUF]
        osems = sems[_NBUF:]
        wid = lax.axis_index("s") * _NC + lax.axis_index("c")
        base = wid * _BPW
        b0 = wid * (_BATCH // _NW)

        pltpu.sync_copy(idx_hbm.at[pl.ds(base, _BPW)], idx_v)

        def start_gather(i, b):
            pltpu.async_copy(
                table_hbm.at[idx_v.at[pl.ds(i * _CH, _CH)]],
                rows_v.at[b], gsems[b])

        def wait_gather(b):
            pltpu.make_async_copy(
                table_hbm.at[idx_v.at[pl.ds(0, _CH)]],
                rows_v.at[b], gsems[b]).wait()

        def start_store(i, b):
            pltpu.async_copy(
                rows_v.at[b], out_hbm.at[b0 + i, :, pl.ds(0, _D)], osems[b])

        def wait_store(b):
            pltpu.make_async_copy(
                rows_v.at[b], out_hbm.at[b0, :, pl.ds(0, _D)], osems[b]).wait()

        for b in range(_NBUF):
            start_gather(b, b)

        @pl.loop(0, _NGRP)
        def _grp(g):
            i0 = g * _NBUF
            for b in range(_NBUF):
                wait_gather(b)
                start_store(i0 + b, b)
            for b in range(_NBUF):
                wait_store(b)
                start_gather(i0 + b + _NBUF, b)

        i0 = _NGRP * _NBUF
        for b in range(_NBUF):
            wait_gather(b)
            start_store(i0 + b, b)
        for b in range(_NBUF):
            wait_store(b)

    return emb


_emb = _make_kernel()


@jax.jit
def kernel(x, table):
    idx = x.reshape(_B).astype(jnp.int32) * 2
    tab2m = jnp.pad(table, ((0, 0), (0, _DP - _D))).reshape(2 * _V, _D)
    out128 = _emb(idx, tab2m)
    return out128[:, :, :_D]
